# baseline (speedup 1.0000x reference)
import functools

import jax
import jax.numpy as jnp
from jax import lax
from jax.experimental import pallas as pl
from jax.experimental.pallas import tpu as pltpu
from jax.experimental.pallas import tpu_sc as plsc

_B, _N, _K = 2, 8192, 16
_NC, _NS = 2, 16
_P = 2048
_GU = 4
_NEG = -3.0e38


def _split(x):
    hi = x.astype(jnp.bfloat16)
    lo = (x - hi.astype(jnp.float32)).astype(jnp.bfloat16)
    return hi, lo


def _dot3(w, fhi, flo):
    whi, wlo = _split(w)
    d = lambda a, b: jnp.dot(a, b, preferred_element_type=jnp.float32)
    return d(whi, fhi) + (d(whi, flo) + d(wlo, fhi))


def _pack2(e, o):
    lo = lax.bitcast_convert_type(e.astype(jnp.bfloat16), jnp.uint16)
    hi = lax.bitcast_convert_type(o.astype(jnp.bfloat16), jnp.uint16)
    return lo.astype(jnp.int32) | (hi.astype(jnp.int32) << 16)


def _tc_proj_body(f_ref, wc_ref, wne_ref, wno_ref, b_ref, a_ref, bv_ref):
    fhi, flo = _split(f_ref[0])
    a_ref[0] = _dot3(wc_ref[...], fhi, flo) + b_ref[...]
    bv_ref[0] = _pack2(
        _dot3(wne_ref[...], fhi, flo),
        _dot3(wno_ref[...], fhi, flo),
    )


def _tc_proj(featT, wc, wn, bias):
    _, cp, n = featT.shape
    op = wc.shape[0]
    return pl.pallas_call(
        _tc_proj_body,
        grid=(_B,),
        in_specs=[
            pl.BlockSpec((1, cp, n), lambda b: (b, 0, 0)),
            pl.BlockSpec((op, cp), lambda b: (0, 0)),
            pl.BlockSpec((op // 2, cp), lambda b: (0, 0)),
            pl.BlockSpec((op // 2, cp), lambda b: (0, 0)),
            pl.BlockSpec((op, 1), lambda b: (0, 0)),
        ],
        out_specs=[
            pl.BlockSpec((1, op, n), lambda b: (b, 0, 0)),
            pl.BlockSpec((1, op // 2, n), lambda b: (b, 0, 0)),
        ],
        out_shape=[
            jax.ShapeDtypeStruct((_B, op, n), jnp.float32),
            jax.ShapeDtypeStruct((_B, op // 2, n), jnp.int32),
        ],
    )(featT, wc, wn[0::2], wn[1::2], bias)


def _tc_proj4_body(f_ref, x_ref, m_ref, wcf_ref, wcx_ref, wcm_ref,
                   wne_ref, wnxe_ref, wnme_ref,
                   wno_ref, wnxo_ref, wnmo_ref, b_ref, a_ref, bv_ref):
    fhi, flo = _split(f_ref[0])
    xhi, xlo = _split(x_ref[0])
    mhi, mlo = _split(m_ref[0])
    a_ref[0] = (
        _dot3(wcf_ref[...], fhi, flo)
        + _dot3(wcx_ref[...], xhi, xlo)
        + (_dot3(wcm_ref[...], mhi, mlo) + b_ref[...])
    )
    bv_e = (
        _dot3(wne_ref[...], fhi, flo)
        + _dot3(wnxe_ref[...], xhi, xlo)
        + _dot3(wnme_ref[...], mhi, mlo)
    )
    bv_o = (
        _dot3(wno_ref[...], fhi, flo)
        + _dot3(wnxo_ref[...], xhi, xlo)
        + _dot3(wnmo_ref[...], mhi, mlo)
    )
    bv_ref[0] = _pack2(bv_e, bv_o)


def _tc_proj4(featT, xyzT, msgc, wcf, wcx, wcm, wnf, wnx, wnm, bias):
    _, cp, n = featT.shape
    cx = xyzT.shape[1]
    op = wcf.shape[0]
    hp = op // 2
    return pl.pallas_call(
        _tc_proj4_body,
        grid=(_B,),
        in_specs=[
            pl.BlockSpec((1, cp, n), lambda b: (b, 0, 0)),
            pl.BlockSpec((1, cx, n), lambda b: (b, 0, 0)),
            pl.BlockSpec((1, 16, 1), lambda b: (b, 0, 0)),
            pl.BlockSpec((op, cp), lambda b: (0, 0)),
            pl.BlockSpec((op, cx), lambda b: (0, 0)),
            pl.BlockSpec((op, 16), lambda b: (0, 0)),
            pl.BlockSpec((hp, cp), lambda b: (0, 0)),
            pl.BlockSpec((hp, cx), lambda b: (0, 0)),
            pl.BlockSpec((hp, 16), lambda b: (0, 0)),
            pl.BlockSpec((hp, cp), lambda b: (0, 0)),
            pl.BlockSpec((hp, cx), lambda b: (0, 0)),
            pl.BlockSpec((hp, 16), lambda b: (0, 0)),
            pl.BlockSpec((op, 1), lambda b: (0, 0)),
        ],
        out_specs=[
            pl.BlockSpec((1, op, n), lambda b: (b, 0, 0)),
            pl.BlockSpec((1, op // 2, n), lambda b: (b, 0, 0)),
        ],
        out_shape=[
            jax.ShapeDtypeStruct((_B, op, n), jnp.float32),
            jax.ShapeDtypeStruct((_B, op // 2, n), jnp.int32),
        ],
    )(featT, xyzT, msgc, wcf, wcx, wcm,
      wnf[0::2], wnx[0::2], wnm[0::2],
      wnf[1::2], wnx[1::2], wnm[1::2], bias)


def _sc_gathermax_body(ppt, active, relu,
                       at_hbm, bvp_hbm, idxp_hbm, out_hbm, *scratch):
    bv_refs = scratch[:ppt]
    a_v, idx_v, out_v = scratch[ppt:ppt + 3]
    sems_i = scratch[ppt + 3:ppt + 5]
    sems_o = scratch[ppt + 5:ppt + 7]
    b = lax.axis_index("c")
    s = lax.axis_index("s")
    cpt = 2 * ppt
    kp = _K // 2

    @pl.when(s < active)
    def _():
        p0 = s * ppt
        nchunk = _N // _P
        copies_i = [None, None]
        copies_o = [None, None]
        copies_i[0] = pltpu.async_copy(
            idxp_hbm.at[b, :, pl.ds(0, _P)], idx_v.at[0], sems_i[0]
        )
        for pi in range(ppt):
            pltpu.sync_copy(bvp_hbm.at[b, p0 + pi], bv_refs[pi])
        pltpu.sync_copy(at_hbm.at[b, pl.ds(s * cpt, cpt), :], a_v)
        for j in range(nchunk):
            buf = j % 2
            if j + 1 < nchunk:
                copies_i[1 - buf] = pltpu.async_copy(
                    idxp_hbm.at[b, :, pl.ds((j + 1) * _P, _P)],
                    idx_v.at[1 - buf], sems_i[1 - buf],
                )
            copies_i[buf].wait()
            if copies_o[buf] is not None:
                copies_o[buf].wait()

            @plsc.parallel_loop(0, _P // 16, 1, unroll=_GU)
            def _g_loop(g, j=j, buf=buf):
                ivs = []
                for k in range(kp):
                    w = idx_v[buf, k, pl.ds(g * 16, 16)]
                    w16 = plsc.bitcast(w, jnp.int16)
                    i_lo, i_hi = plsc.unpack(
                        w16, format=plsc.PackFormat.INTERLEAVED
                    )
                    ivs += [i_lo, i_hi]
                for pi in range(ppt):
                    m = jnp.full((32,), _NEG, jnp.bfloat16)
                    for k in range(_K):
                        vals = plsc.load_gather(bv_refs[pi], [ivs[k]])
                        m = jnp.maximum(
                            m, plsc.bitcast(vals, jnp.bfloat16)
                        )
                    m_lo, m_hi = plsc.unpack(
                        m, format=plsc.PackFormat.INTERLEAVED
                    )
                    for ci, mm in ((2 * pi, m_lo), (2 * pi + 1, m_hi)):
                        r = mm + a_v[ci, pl.ds(j * _P + g * 16, 16)]
                        if relu:
                            r = jnp.maximum(r, 0.0)
                        out_v[buf, ci, pl.ds(g * 16, 16)] = r
            copies_o[buf] = pltpu.async_copy(
                out_v.at[buf],
                out_hbm.at[b, pl.ds(s * cpt, cpt), pl.ds(j * _P, _P)],
                sems_o[buf],
            )
        for c in copies_o:
            if c is not None:
                c.wait()


@functools.lru_cache(maxsize=None)
def _sc_gathermax(op, relu):
    pairs = op // 2
    ppt = max(1, pairs // _NS)
    active = pairs // ppt
    cpt = 2 * ppt
    mesh = plsc.VectorSubcoreMesh(
        core_axis_name="c", subcore_axis_name="s",
        num_cores=_NC, num_subcores=_NS,
    )
    return pl.kernel(
        functools.partial(_sc_gathermax_body, ppt, active, relu),
        out_type=jax.ShapeDtypeStruct((_B, op, _N), jnp.float32),
        mesh=mesh,
        compiler_params=pltpu.CompilerParams(needs_layout_passes=False),
        scratch_types=(
            [pltpu.VMEM((_N,), jnp.int32) for _ in range(ppt)]
            + [
                pltpu.VMEM((cpt, _N), jnp.float32),
                pltpu.VMEM((2, _K // 2, _P), jnp.int32),
                pltpu.VMEM((2, cpt, _P), jnp.float32),
                pltpu.SemaphoreType.DMA,
                pltpu.SemaphoreType.DMA,
                pltpu.SemaphoreType.DMA,
                pltpu.SemaphoreType.DMA,
            ]
        ),
    )


def _edge_layer(featT, idxP, w, bias, cin, op, relu):
    cp = featT.shape[1]
    wc = w[:, :cin] - w[:, cin:]
    wn = w[:, cin:]
    if cp > cin:
        pad = ((0, 0), (0, cp - cin))
        wc = jnp.pad(wc, pad)
        wn = jnp.pad(wn, pad)
    if op > w.shape[0]:
        wc = jnp.pad(wc, ((0, op - w.shape[0]), (0, 0)))
        wn = jnp.pad(wn, ((0, op - w.shape[0]), (0, 0)))
        bias = jnp.pad(bias, (0, op - bias.shape[0]))
    a, bvp = _tc_proj(featT, wc, wn, bias[:, None])
    return _sc_gathermax(op, relu)(a, bvp, idxP)


def kernel(xyz, msg, W1, b1, W2, b2, W3, b3, Wc, bc, Wf, bf, idx):
    idxT = jnp.swapaxes(idx, 1, 2)
    idxP = idxT[:, 0::2, :] | (idxT[:, 1::2, :] << 16)
    xyzT = jnp.swapaxes(xyz, 1, 2)
    xyzT8 = jnp.pad(xyzT, ((0, 0), (0, 5), (0, 0)))

    feat = _edge_layer(xyzT8, idxP, W1, b1, cin=3, op=64, relu=True)
    feat = _edge_layer(feat, idxP, W2, b2, cin=64, op=64, relu=True)
    feat = _edge_layer(feat, idxP, W3, b3, cin=64, op=64, relu=True)

    wcc = Wc[:, :83] - Wc[:, 83:]
    wnc = Wc[:, 83:]
    wcx = jnp.pad(wcc[:, 64:67], ((0, 0), (0, 5)))
    wnx = jnp.pad(wnc[:, 64:67], ((0, 0), (0, 5)))
    a4, bv4 = _tc_proj4(
        feat, xyzT8, msg[:, :, None],
        wcc[:, :64], wcx, wcc[:, 67:83],
        wnc[:, :64], wnx, wnc[:, 67:83],
        bc[:, None],
    )
    feat = _sc_gathermax(64, True)(a4, bv4, idxP)

    feat = _edge_layer(feat, idxP, Wf, bf, cin=64, op=8, relu=False)
    return jnp.swapaxes(feat[:, :3, :], 1, 2)


# speedup vs baseline: 1.1602x; 1.1602x over previous
import functools

import jax
import jax.numpy as jnp
from jax import lax
from jax.experimental import pallas as pl
from jax.experimental.pallas import tpu as pltpu
from jax.experimental.pallas import tpu_sc as plsc

_B, _N, _K = 2, 8192, 16
_NC, _NS = 2, 16
_P = 2048
_GU = 1
_NEG = -3.0e38


def _split(x):
    hi = x.astype(jnp.bfloat16)
    lo = (x - hi.astype(jnp.float32)).astype(jnp.bfloat16)
    return hi, lo


def _dot3(w, fhi, flo):
    whi, wlo = _split(w)
    d = lambda a, b: jnp.dot(a, b, preferred_element_type=jnp.float32)
    return d(whi, fhi) + (d(whi, flo) + d(wlo, fhi))


def _pack2(e, o):
    lo = lax.bitcast_convert_type(e.astype(jnp.bfloat16), jnp.uint16)
    hi = lax.bitcast_convert_type(o.astype(jnp.bfloat16), jnp.uint16)
    return lo.astype(jnp.int32) | (hi.astype(jnp.int32) << 16)


def _tc_proj_body(f_ref, wc_ref, wne_ref, wno_ref, b_ref, a_ref, bv_ref):
    fhi, flo = _split(f_ref[0])
    a_ref[0] = _dot3(wc_ref[...], fhi, flo) + b_ref[...]
    bv_ref[0] = _pack2(
        _dot3(wne_ref[...], fhi, flo),
        _dot3(wno_ref[...], fhi, flo),
    )


def _tc_proj(featT, wc, wn, bias):
    _, cp, n = featT.shape
    op = wc.shape[0]
    return pl.pallas_call(
        _tc_proj_body,
        grid=(_B,),
        in_specs=[
            pl.BlockSpec((1, cp, n), lambda b: (b, 0, 0)),
            pl.BlockSpec((op, cp), lambda b: (0, 0)),
            pl.BlockSpec((op // 2, cp), lambda b: (0, 0)),
            pl.BlockSpec((op // 2, cp), lambda b: (0, 0)),
            pl.BlockSpec((op, 1), lambda b: (0, 0)),
        ],
        out_specs=[
            pl.BlockSpec((1, op, n), lambda b: (b, 0, 0)),
            pl.BlockSpec((1, op // 2, n), lambda b: (b, 0, 0)),
        ],
        out_shape=[
            jax.ShapeDtypeStruct((_B, op, n), jnp.float32),
            jax.ShapeDtypeStruct((_B, op // 2, n), jnp.int32),
        ],
    )(featT, wc, wn[0::2], wn[1::2], bias)


def _tc_proj4_body(f_ref, x_ref, m_ref, wcf_ref, wcx_ref, wcm_ref,
                   wne_ref, wnxe_ref, wnme_ref,
                   wno_ref, wnxo_ref, wnmo_ref, b_ref, a_ref, bv_ref):
    fhi, flo = _split(f_ref[0])
    xhi, xlo = _split(x_ref[0])
    mhi, mlo = _split(m_ref[0])
    a_ref[0] = (
        _dot3(wcf_ref[...], fhi, flo)
        + _dot3(wcx_ref[...], xhi, xlo)
        + (_dot3(wcm_ref[...], mhi, mlo) + b_ref[...])
    )
    bv_e = (
        _dot3(wne_ref[...], fhi, flo)
        + _dot3(wnxe_ref[...], xhi, xlo)
        + _dot3(wnme_ref[...], mhi, mlo)
    )
    bv_o = (
        _dot3(wno_ref[...], fhi, flo)
        + _dot3(wnxo_ref[...], xhi, xlo)
        + _dot3(wnmo_ref[...], mhi, mlo)
    )
    bv_ref[0] = _pack2(bv_e, bv_o)


def _tc_proj4(featT, xyzT, msgc, wcf, wcx, wcm, wnf, wnx, wnm, bias):
    _, cp, n = featT.shape
    cx = xyzT.shape[1]
    op = wcf.shape[0]
    hp = op // 2
    return pl.pallas_call(
        _tc_proj4_body,
        grid=(_B,),
        in_specs=[
            pl.BlockSpec((1, cp, n), lambda b: (b, 0, 0)),
            pl.BlockSpec((1, cx, n), lambda b: (b, 0, 0)),
            pl.BlockSpec((1, 16, 1), lambda b: (b, 0, 0)),
            pl.BlockSpec((op, cp), lambda b: (0, 0)),
            pl.BlockSpec((op, cx), lambda b: (0, 0)),
            pl.BlockSpec((op, 16), lambda b: (0, 0)),
            pl.BlockSpec((hp, cp), lambda b: (0, 0)),
            pl.BlockSpec((hp, cx), lambda b: (0, 0)),
            pl.BlockSpec((hp, 16), lambda b: (0, 0)),
            pl.BlockSpec((hp, cp), lambda b: (0, 0)),
            pl.BlockSpec((hp, cx), lambda b: (0, 0)),
            pl.BlockSpec((hp, 16), lambda b: (0, 0)),
            pl.BlockSpec((op, 1), lambda b: (0, 0)),
        ],
        out_specs=[
            pl.BlockSpec((1, op, n), lambda b: (b, 0, 0)),
            pl.BlockSpec((1, op // 2, n), lambda b: (b, 0, 0)),
        ],
        out_shape=[
            jax.ShapeDtypeStruct((_B, op, n), jnp.float32),
            jax.ShapeDtypeStruct((_B, op // 2, n), jnp.int32),
        ],
    )(featT, xyzT, msgc, wcf, wcx, wcm,
      wnf[0::2], wnx[0::2], wnm[0::2],
      wnf[1::2], wnx[1::2], wnm[1::2], bias)


def _sc_gathermax_body(ppt, active, relu,
                       at_hbm, bvp_hbm, idxp_hbm, out_hbm, *scratch):
    bv_refs = scratch[:ppt]
    a_v, idx_v, out_v = scratch[ppt:ppt + 3]
    sems_i = scratch[ppt + 3:ppt + 5]
    sems_o = scratch[ppt + 5:ppt + 7]
    b = lax.axis_index("c")
    s = lax.axis_index("s")
    cpt = 2 * ppt
    kp = _K // 2

    @pl.when(s < active)
    def _():
        p0 = s * ppt
        nchunk = _N // _P
        copies_i = [None, None]
        copies_o = [None, None]
        copies_i[0] = pltpu.async_copy(
            idxp_hbm.at[b, :, pl.ds(0, _P)], idx_v.at[0], sems_i[0]
        )
        for pi in range(ppt):
            pltpu.sync_copy(bvp_hbm.at[b, p0 + pi], bv_refs[pi])
        pltpu.sync_copy(at_hbm.at[b, pl.ds(s * cpt, cpt), :], a_v)
        for j in range(nchunk):
            buf = j % 2
            if j + 1 < nchunk:
                copies_i[1 - buf] = pltpu.async_copy(
                    idxp_hbm.at[b, :, pl.ds((j + 1) * _P, _P)],
                    idx_v.at[1 - buf], sems_i[1 - buf],
                )
            copies_i[buf].wait()
            if copies_o[buf] is not None:
                copies_o[buf].wait()

            @plsc.parallel_loop(0, _P // 16, 1, unroll=_GU)
            def _g_loop(g, j=j, buf=buf):
                ivs = []
                for k in range(kp):
                    w = idx_v[buf, k, pl.ds(g * 16, 16)]
                    w16 = plsc.bitcast(w, jnp.int16)
                    i_lo, i_hi = plsc.unpack(
                        w16, format=plsc.PackFormat.INTERLEAVED
                    )
                    ivs += [i_lo, i_hi]
                for pi in range(ppt):
                    m = jnp.full((32,), _NEG, jnp.bfloat16)
                    for k in range(_K):
                        vals = plsc.load_gather(bv_refs[pi], [ivs[k]])
                        m = jnp.maximum(
                            m, plsc.bitcast(vals, jnp.bfloat16)
                        )
                    m_lo, m_hi = plsc.unpack(
                        m, format=plsc.PackFormat.INTERLEAVED
                    )
                    for ci, mm in ((2 * pi, m_lo), (2 * pi + 1, m_hi)):
                        r = mm + a_v[ci, pl.ds(j * _P + g * 16, 16)]
                        if relu:
                            r = jnp.maximum(r, 0.0)
                        out_v[buf, ci, pl.ds(g * 16, 16)] = r
            copies_o[buf] = pltpu.async_copy(
                out_v.at[buf],
                out_hbm.at[b, pl.ds(s * cpt, cpt), pl.ds(j * _P, _P)],
                sems_o[buf],
            )
        for c in copies_o:
            if c is not None:
                c.wait()


@functools.lru_cache(maxsize=None)
def _sc_gathermax(op, relu):
    pairs = op // 2
    ppt = max(1, pairs // _NS)
    active = pairs // ppt
    cpt = 2 * ppt
    mesh = plsc.VectorSubcoreMesh(
        core_axis_name="c", subcore_axis_name="s",
        num_cores=_NC, num_subcores=_NS,
    )
    return pl.kernel(
        functools.partial(_sc_gathermax_body, ppt, active, relu),
        out_type=jax.ShapeDtypeStruct((_B, op, _N), jnp.float32),
        mesh=mesh,
        compiler_params=pltpu.CompilerParams(needs_layout_passes=False),
        scratch_types=(
            [pltpu.VMEM((_N,), jnp.int32) for _ in range(ppt)]
            + [
                pltpu.VMEM((cpt, _N), jnp.float32),
                pltpu.VMEM((2, _K // 2, _P), jnp.int32),
                pltpu.VMEM((2, cpt, _P), jnp.float32),
                pltpu.SemaphoreType.DMA,
                pltpu.SemaphoreType.DMA,
                pltpu.SemaphoreType.DMA,
                pltpu.SemaphoreType.DMA,
            ]
        ),
    )


def _edge_layer(featT, idxP, w, bias, cin, op, relu):
    cp = featT.shape[1]
    wc = w[:, :cin] - w[:, cin:]
    wn = w[:, cin:]
    if cp > cin:
        pad = ((0, 0), (0, cp - cin))
        wc = jnp.pad(wc, pad)
        wn = jnp.pad(wn, pad)
    if op > w.shape[0]:
        wc = jnp.pad(wc, ((0, op - w.shape[0]), (0, 0)))
        wn = jnp.pad(wn, ((0, op - w.shape[0]), (0, 0)))
        bias = jnp.pad(bias, (0, op - bias.shape[0]))
    a, bvp = _tc_proj(featT, wc, wn, bias[:, None])
    return _sc_gathermax(op, relu)(a, bvp, idxP)


def kernel(xyz, msg, W1, b1, W2, b2, W3, b3, Wc, bc, Wf, bf, idx):
    idxT = jnp.swapaxes(idx, 1, 2)
    idxP = idxT[:, 0::2, :] | (idxT[:, 1::2, :] << 16)
    xyzT = jnp.swapaxes(xyz, 1, 2)
    xyzT8 = jnp.pad(xyzT, ((0, 0), (0, 5), (0, 0)))

    feat = _edge_layer(xyzT8, idxP, W1, b1, cin=3, op=64, relu=True)
    feat = _edge_layer(feat, idxP, W2, b2, cin=64, op=64, relu=True)
    feat = _edge_layer(feat, idxP, W3, b3, cin=64, op=64, relu=True)

    wcc = Wc[:, :83] - Wc[:, 83:]
    wnc = Wc[:, 83:]
    wcx = jnp.pad(wcc[:, 64:67], ((0, 0), (0, 5)))
    wnx = jnp.pad(wnc[:, 64:67], ((0, 0), (0, 5)))
    a4, bv4 = _tc_proj4(
        feat, xyzT8, msg[:, :, None],
        wcc[:, :64], wcx, wcc[:, 67:83],
        wnc[:, :64], wnx, wnc[:, 67:83],
        bc[:, None],
    )
    feat = _sc_gathermax(64, True)(a4, bv4, idxP)

    feat = _edge_layer(feat, idxP, Wf, bf, cin=64, op=8, relu=False)
    return jnp.swapaxes(feat[:, :3, :], 1, 2)


# speedup vs baseline: 1.2075x; 1.0408x over previous
import functools

import jax
import jax.numpy as jnp
from jax import lax
from jax.experimental import pallas as pl
from jax.experimental.pallas import tpu as pltpu
from jax.experimental.pallas import tpu_sc as plsc

_B, _N, _K = 2, 8192, 16
_NC, _NS = 2, 16
_P = 2048
_GU = 1
_NEG = -3.0e38


def _split(x):
    hi = x.astype(jnp.bfloat16)
    lo = (x - hi.astype(jnp.float32)).astype(jnp.bfloat16)
    return hi, lo


def _dot3(w, fhi, flo):
    whi, wlo = _split(w)
    d = lambda a, b: jnp.dot(a, b, preferred_element_type=jnp.float32)
    return d(whi, fhi) + (d(whi, flo) + d(wlo, fhi))


def _pack2(e, o):
    lo = lax.bitcast_convert_type(e.astype(jnp.bfloat16), jnp.uint16)
    hi = lax.bitcast_convert_type(o.astype(jnp.bfloat16), jnp.uint16)
    return lo.astype(jnp.int32) | (hi.astype(jnp.int32) << 16)


def _tc_proj_body(f_ref, wc_ref, wne_ref, wno_ref, b_ref, a_ref, bv_ref):
    fhi, flo = _split(f_ref[0])
    a_ref[0] = _dot3(wc_ref[...], fhi, flo) + b_ref[...]
    bv_ref[0] = _pack2(
        _dot3(wne_ref[...], fhi, flo),
        _dot3(wno_ref[...], fhi, flo),
    )


def _tc_proj(featT, wc, wn, bias):
    _, cp, n = featT.shape
    op = wc.shape[0]
    return pl.pallas_call(
        _tc_proj_body,
        grid=(_B,),
        in_specs=[
            pl.BlockSpec((1, cp, n), lambda b: (b, 0, 0)),
            pl.BlockSpec((op, cp), lambda b: (0, 0)),
            pl.BlockSpec((op // 2, cp), lambda b: (0, 0)),
            pl.BlockSpec((op // 2, cp), lambda b: (0, 0)),
            pl.BlockSpec((op, 1), lambda b: (0, 0)),
        ],
        out_specs=[
            pl.BlockSpec((1, op, n), lambda b: (b, 0, 0)),
            pl.BlockSpec((1, op // 2, n), lambda b: (b, 0, 0)),
        ],
        out_shape=[
            jax.ShapeDtypeStruct((_B, op, n), jnp.float32),
            jax.ShapeDtypeStruct((_B, op // 2, n), jnp.int32),
        ],
    )(featT, wc, wn[0::2], wn[1::2], bias)


def _tc_proj4_body(f_ref, x_ref, m_ref, wcf_ref, wcx_ref, wcm_ref,
                   wne_ref, wnxe_ref, wnme_ref,
                   wno_ref, wnxo_ref, wnmo_ref, b_ref, a_ref, bv_ref):
    fhi, flo = _split(f_ref[0])
    xhi, xlo = _split(x_ref[0])
    mhi, mlo = _split(m_ref[0])
    a_ref[0] = (
        _dot3(wcf_ref[...], fhi, flo)
        + _dot3(wcx_ref[...], xhi, xlo)
        + (_dot3(wcm_ref[...], mhi, mlo) + b_ref[...])
    )
    bv_e = (
        _dot3(wne_ref[...], fhi, flo)
        + _dot3(wnxe_ref[...], xhi, xlo)
        + _dot3(wnme_ref[...], mhi, mlo)
    )
    bv_o = (
        _dot3(wno_ref[...], fhi, flo)
        + _dot3(wnxo_ref[...], xhi, xlo)
        + _dot3(wnmo_ref[...], mhi, mlo)
    )
    bv_ref[0] = _pack2(bv_e, bv_o)


def _tc_proj4(featT, xyzT, msgc, wcf, wcx, wcm, wnf, wnx, wnm, bias):
    _, cp, n = featT.shape
    cx = xyzT.shape[1]
    op = wcf.shape[0]
    hp = op // 2
    return pl.pallas_call(
        _tc_proj4_body,
        grid=(_B,),
        in_specs=[
            pl.BlockSpec((1, cp, n), lambda b: (b, 0, 0)),
            pl.BlockSpec((1, cx, n), lambda b: (b, 0, 0)),
            pl.BlockSpec((1, 16, 1), lambda b: (b, 0, 0)),
            pl.BlockSpec((op, cp), lambda b: (0, 0)),
            pl.BlockSpec((op, cx), lambda b: (0, 0)),
            pl.BlockSpec((op, 16), lambda b: (0, 0)),
            pl.BlockSpec((hp, cp), lambda b: (0, 0)),
            pl.BlockSpec((hp, cx), lambda b: (0, 0)),
            pl.BlockSpec((hp, 16), lambda b: (0, 0)),
            pl.BlockSpec((hp, cp), lambda b: (0, 0)),
            pl.BlockSpec((hp, cx), lambda b: (0, 0)),
            pl.BlockSpec((hp, 16), lambda b: (0, 0)),
            pl.BlockSpec((op, 1), lambda b: (0, 0)),
        ],
        out_specs=[
            pl.BlockSpec((1, op, n), lambda b: (b, 0, 0)),
            pl.BlockSpec((1, op // 2, n), lambda b: (b, 0, 0)),
        ],
        out_shape=[
            jax.ShapeDtypeStruct((_B, op, n), jnp.float32),
            jax.ShapeDtypeStruct((_B, op // 2, n), jnp.int32),
        ],
    )(featT, xyzT, msgc, wcf, wcx, wcm,
      wnf[0::2], wnx[0::2], wnm[0::2],
      wnf[1::2], wnx[1::2], wnm[1::2], bias)


def _sc_gathermax_body(ppt, npg, psplit, relu,
                       at_hbm, bvp_hbm, idxp_hbm, out_hbm, *scratch):
    bv_refs = scratch[:ppt]
    a_v, idx_v, out_v = scratch[ppt:ppt + 3]
    sems_i = scratch[ppt + 3:ppt + 5]
    sems_o = scratch[ppt + 5:ppt + 7]
    b = lax.axis_index("c")
    s = lax.axis_index("s")
    cpt = 2 * ppt
    kp = _K // 2
    pspan = _N // psplit

    @pl.when(s < npg * psplit)
    def _():
        pg = s % npg
        pc = s // npg
        p0 = pg * ppt
        pbase = pc * pspan
        nchunk = pspan // _P
        copies_i = [None, None]
        copies_o = [None, None]
        copies_i[0] = pltpu.async_copy(
            idxp_hbm.at[b, :, pl.ds(pbase, _P)], idx_v.at[0], sems_i[0]
        )
        for pi in range(ppt):
            pltpu.sync_copy(bvp_hbm.at[b, p0 + pi], bv_refs[pi])
        pltpu.sync_copy(
            at_hbm.at[b, pl.ds(pg * cpt, cpt), pl.ds(pbase, pspan)], a_v
        )
        for j in range(nchunk):
            buf = j % 2
            if j + 1 < nchunk:
                copies_i[1 - buf] = pltpu.async_copy(
                    idxp_hbm.at[b, :, pl.ds(pbase + (j + 1) * _P, _P)],
                    idx_v.at[1 - buf], sems_i[1 - buf],
                )
            copies_i[buf].wait()
            if copies_o[buf] is not None:
                copies_o[buf].wait()

            @plsc.parallel_loop(0, _P // 16, 1, unroll=_GU)
            def _g_loop(g, j=j, buf=buf):
                ivs = []
                for k in range(kp):
                    w = idx_v[buf, k, pl.ds(g * 16, 16)]
                    w16 = plsc.bitcast(w, jnp.int16)
                    i_lo, i_hi = plsc.unpack(
                        w16, format=plsc.PackFormat.INTERLEAVED
                    )
                    ivs += [i_lo, i_hi]
                for pi in range(ppt):
                    m = jnp.full((32,), _NEG, jnp.bfloat16)
                    for k in range(_K):
                        vals = plsc.load_gather(bv_refs[pi], [ivs[k]])
                        m = jnp.maximum(
                            m, plsc.bitcast(vals, jnp.bfloat16)
                        )
                    m_lo, m_hi = plsc.unpack(
                        m, format=plsc.PackFormat.INTERLEAVED
                    )
                    for ci, mm in ((2 * pi, m_lo), (2 * pi + 1, m_hi)):
                        r = mm + a_v[ci, pl.ds(j * _P + g * 16, 16)]
                        if relu:
                            r = jnp.maximum(r, 0.0)
                        out_v[buf, ci, pl.ds(g * 16, 16)] = r
            copies_o[buf] = pltpu.async_copy(
                out_v.at[buf],
                out_hbm.at[b, pl.ds(pg * cpt, cpt),
                           pl.ds(pbase + j * _P, _P)],
                sems_o[buf],
            )
        for c in copies_o:
            if c is not None:
                c.wait()


@functools.lru_cache(maxsize=None)
def _sc_gathermax(op, relu):
    pairs = op // 2
    ppt = max(1, pairs // _NS)
    npg = pairs // ppt
    psplit = _NS // npg
    cpt = 2 * ppt
    mesh = plsc.VectorSubcoreMesh(
        core_axis_name="c", subcore_axis_name="s",
        num_cores=_NC, num_subcores=_NS,
    )
    return pl.kernel(
        functools.partial(_sc_gathermax_body, ppt, npg, psplit, relu),
        out_type=jax.ShapeDtypeStruct((_B, op, _N), jnp.float32),
        mesh=mesh,
        compiler_params=pltpu.CompilerParams(needs_layout_passes=False),
        scratch_types=(
            [pltpu.VMEM((_N,), jnp.int32) for _ in range(ppt)]
            + [
                pltpu.VMEM((cpt, _N // (_NS // (pairs // ppt))),
                           jnp.float32),
                pltpu.VMEM((2, _K // 2, _P), jnp.int32),
                pltpu.VMEM((2, cpt, _P), jnp.float32),
                pltpu.SemaphoreType.DMA,
                pltpu.SemaphoreType.DMA,
                pltpu.SemaphoreType.DMA,
                pltpu.SemaphoreType.DMA,
            ]
        ),
    )


def _edge_layer(featT, idxP, w, bias, cin, op, relu):
    cp = featT.shape[1]
    wc = w[:, :cin] - w[:, cin:]
    wn = w[:, cin:]
    if cp > cin:
        pad = ((0, 0), (0, cp - cin))
        wc = jnp.pad(wc, pad)
        wn = jnp.pad(wn, pad)
    if op > w.shape[0]:
        wc = jnp.pad(wc, ((0, op - w.shape[0]), (0, 0)))
        wn = jnp.pad(wn, ((0, op - w.shape[0]), (0, 0)))
        bias = jnp.pad(bias, (0, op - bias.shape[0]))
    a, bvp = _tc_proj(featT, wc, wn, bias[:, None])
    return _sc_gathermax(op, relu)(a, bvp, idxP)


def kernel(xyz, msg, W1, b1, W2, b2, W3, b3, Wc, bc, Wf, bf, idx):
    idxT = jnp.swapaxes(idx, 1, 2)
    idxP = idxT[:, 0::2, :] | (idxT[:, 1::2, :] << 16)
    xyzT = jnp.swapaxes(xyz, 1, 2)
    xyzT8 = jnp.pad(xyzT, ((0, 0), (0, 5), (0, 0)))

    feat = _edge_layer(xyzT8, idxP, W1, b1, cin=3, op=64, relu=True)
    feat = _edge_layer(feat, idxP, W2, b2, cin=64, op=64, relu=True)
    feat = _edge_layer(feat, idxP, W3, b3, cin=64, op=64, relu=True)

    wcc = Wc[:, :83] - Wc[:, 83:]
    wnc = Wc[:, 83:]
    wcx = jnp.pad(wcc[:, 64:67], ((0, 0), (0, 5)))
    wnx = jnp.pad(wnc[:, 64:67], ((0, 0), (0, 5)))
    a4, bv4 = _tc_proj4(
        feat, xyzT8, msg[:, :, None],
        wcc[:, :64], wcx, wcc[:, 67:83],
        wnc[:, :64], wnx, wnc[:, 67:83],
        bc[:, None],
    )
    feat = _sc_gathermax(64, True)(a4, bv4, idxP)

    feat = _edge_layer(feat, idxP, Wf, bf, cin=64, op=8, relu=False)
    return jnp.swapaxes(feat[:, :3, :], 1, 2)
